# baseline (speedup 1.0000x reference)
import jax
import jax.numpy as jnp
from jax import lax
from jax.experimental import pallas as pl
from jax.experimental.pallas import tpu as pltpu
from jax.experimental.pallas import tpu_sc as plsc

_B, _C, _H, _W = 16, 3, 512, 512
_N = _B * _H * _W
_NB = 8
_PB = _H * _W // _NB
_RB = _PB // 128

_NW = 32
_PPW = _N // _NW
_CH = 2048
_NCHUNK = _PPW // _CH
_KG = _CH // 128


def _stage_a_body(w_ref, i0_ref, i1_ref, sel_ref,
                  wa_ref, wb_ref, wc_ref, wd_ref):
    b = pl.program_id(0)
    hb = pl.program_id(1)
    X = w_ref[0, 0]
    Y = w_ref[0, 1]
    T = w_ref[0, 2]
    r = lax.broadcasted_iota(jnp.int32, (_RB, 128), 0)
    l = lax.broadcasted_iota(jnp.int32, (_RB, 128), 1)
    xxi = (r & 3) * 128 + l
    yyi = hb * (_PB // _W) + (r >> 2)
    xx = xxi.astype(jnp.float32)
    yy = yyi.astype(jnp.float32)
    sm = jnp.where(jnp.abs(T) >= 1e-07, jnp.float32(0.0), jnp.float32(1e-06))
    Tt = T + sm
    v1 = X / Tt
    v2 = Y / Tt
    vgx = xx + (v1 - xx)
    vgy = yy + (v2 - yy)
    x0i = jnp.floor(vgx).astype(jnp.int32)
    y0i = jnp.floor(vgy).astype(jnp.int32)
    x0 = jnp.clip(x0i, 0, _W - 1)
    x1 = jnp.clip(x0i + 1, 0, _W - 1)
    y0 = jnp.clip(y0i, 0, _H - 1)
    y1 = jnp.clip(y0i + 1, 0, _H - 1)
    x0f = x0.astype(jnp.float32)
    x1f = x1.astype(jnp.float32)
    y0f = y0.astype(jnp.float32)
    y1f = y1.astype(jnp.float32)
    Xa = x1f - vgx
    Xc = vgx - x0f
    Ya = y1f - vgy
    Yb = vgy - y0f
    wa_ref[...] = Xa * Ya
    wb_ref[...] = Xa * Yb
    wc_ref[...] = Xc * Ya
    wd_ref[...] = Xc * Yb
    base = b * (_H * _W)
    i0_ref[...] = (base + y0 * _W) + x0
    i1_ref[...] = (base + y1 * _W) + x0
    sel_ref[...] = x1 - x0


def _stage_a(warped_r):
    i_sd = jax.ShapeDtypeStruct((_N // 128, 128), jnp.int32)
    f_sd = jax.ShapeDtypeStruct((_N // 128, 128), jnp.float32)
    out_spec = pl.BlockSpec((_RB, 128), lambda b, h: (b * _NB + h, 0))
    return pl.pallas_call(
        _stage_a_body,
        grid=(_B, _NB),
        in_specs=[pl.BlockSpec((1, 3, _RB, 128), lambda b, h: (b, 0, h, 0))],
        out_specs=[out_spec] * 7,
        out_shape=[i_sd, i_sd, i_sd, f_sd, f_sd, f_sd, f_sd],
    )(warped_r)


def _stage_b_body(table, i0, i1, g0, g1,
                  i0v, i1v, gv0, gv1, gsem, isem, osem):
    wid = lax.axis_index("s") * 2 + lax.axis_index("c")

    def chunk_body(i, _):
        row0 = wid * (_PPW // 128) + i * _KG
        p0 = wid * _PPW + i * _CH

        h_in = []
        for src_ref, dst in ((i0, i0v), (i1, i1v)):
            h_in.append(pltpu.make_async_copy(
                src_ref.at[pl.ds(row0, _KG), :], dst, isem))
        for h in h_in:
            h.start()
        for h in h_in:
            h.wait()

        h_g = []
        for idx_v, dst in ((i0v, gv0), (i1v, gv1)):
            for k in range(_KG):
                h_g.append(pltpu.make_async_copy(
                    table.at[idx_v.at[k]],
                    dst.at[pl.ds(k * 128, 128), :], gsem))
        for h in h_g:
            h.start()
        for h in h_g:
            h.wait()

        h_out = []
        for gv, out_ref in ((gv0, g0), (gv1, g1)):
            h_out.append(pltpu.make_async_copy(
                gv, out_ref.at[pl.ds(p0, _CH), :], osem))
        for h in h_out:
            h.start()
        for h in h_out:
            h.wait()
        return 0

    lax.fori_loop(0, _NCHUNK, chunk_body, 0)


def _stage_b(im8, i02, i12):
    mesh = plsc.VectorSubcoreMesh(core_axis_name="c", subcore_axis_name="s")
    g_sd = jax.ShapeDtypeStruct((_N, 8), jnp.float32)
    kern = pl.kernel(
        _stage_b_body,
        out_type=(g_sd, g_sd),
        mesh=mesh,
        scratch_types=[
            pltpu.VMEM((_KG, 128), jnp.int32),
            pltpu.VMEM((_KG, 128), jnp.int32),
            pltpu.VMEM((_CH, 8), jnp.float32),
            pltpu.VMEM((_CH, 8), jnp.float32),
            pltpu.SemaphoreType.DMA,
            pltpu.SemaphoreType.DMA,
            pltpu.SemaphoreType.DMA,
        ],
        compiler_params=pltpu.CompilerParams(use_tc_tiling_on_sc=False),
    )
    return kern(im8, i02, i12)


def _stage_c_body(g0_ref, g1_ref, sel_ref,
                  wa_ref, wb_ref, wc_ref, wd_ref, out_ref):
    w_a = wa_ref[...]
    w_b = wb_ref[...]
    w_c = wc_ref[...]
    w_d = wd_ref[...]
    hi = sel_ref[...] > 0
    for ch in range(_C):
        i_a = g0_ref[ch]
        i_b = g1_ref[ch]
        i_c = jnp.where(hi, g0_ref[ch + 3], i_a)
        i_d = jnp.where(hi, g1_ref[ch + 3], i_b)
        out_ref[0, ch] = ((w_a * i_a + w_b * i_b) + w_c * i_c) + w_d * i_d


def _stage_c(g0p, g1p, sel, wa, wb, wc, wd):
    g_spec = pl.BlockSpec((6, _RB, 128), lambda b, h: (0, b * _NB + h, 0))
    w_spec = pl.BlockSpec((_RB, 128), lambda b, h: (b * _NB + h, 0))
    return pl.pallas_call(
        _stage_c_body,
        grid=(_B, _NB),
        in_specs=[g_spec, g_spec] + [w_spec] * 5,
        out_specs=pl.BlockSpec((1, _C, _RB, 128), lambda b, h: (b, 0, h, 0)),
        out_shape=jax.ShapeDtypeStruct((_B, _C, _NB * _RB, 128), jnp.float32),
    )(g0p, g1p, sel, wa, wb, wc, wd)


def kernel(src, H):
    b, c, h, w = src.shape
    xx = jnp.tile(jnp.arange(w)[None, :], (h, 1))
    yy = jnp.tile(jnp.arange(h)[:, None], (1, w))
    ones = jnp.ones((h, w), dtype=jnp.int32)
    g = jnp.stack([xx, yy, ones], axis=0).astype(jnp.float32)
    grid = jnp.broadcast_to(g[None], (b, 3, h, w))
    warped = jnp.einsum('bij,bjhw->bihw', H.reshape(b, 3, 3), grid)
    warped_r = warped.reshape(b, 3, _NB * _RB, 128)
    i0, i1, sel, wa, wb, wc, wd = _stage_a(warped_r)
    im_flat = src.transpose(0, 2, 3, 1).reshape(-1, c)
    shifted = jnp.concatenate([im_flat[1:], im_flat[:1]], axis=0)
    im8 = jnp.concatenate(
        [im_flat, shifted, jnp.zeros((_N, 2), jnp.float32)], axis=1)
    g0, g1 = _stage_b(im8, i0, i1)
    g0p = g0[:, :6].T.reshape(6, _N // 128, 128)
    g1p = g1[:, :6].T.reshape(6, _N // 128, 128)
    out = _stage_c(g0p, g1p, sel, wa, wb, wc, wd)
    return out.reshape(b, c, h, w)


# speedup vs baseline: 1.0065x; 1.0065x over previous
import jax
import jax.numpy as jnp
from jax import lax
from jax.experimental import pallas as pl
from jax.experimental.pallas import tpu as pltpu
from jax.experimental.pallas import tpu_sc as plsc

_B, _C, _H, _W = 16, 3, 512, 512
_N = _B * _H * _W
_NB = 8
_PB = _H * _W // _NB
_RB = _PB // 128

_NW = 32
_PPW = _N // _NW
_CH = 2048
_NCHUNK = _PPW // _CH
_KG = _CH // 128


def _stage_a_body(w_ref, i0_ref, i1_ref, sel_ref,
                  wa_ref, wb_ref, wc_ref, wd_ref):
    b = pl.program_id(0)
    hb = pl.program_id(1)
    X = w_ref[0, 0]
    Y = w_ref[0, 1]
    T = w_ref[0, 2]
    r = lax.broadcasted_iota(jnp.int32, (_RB, 128), 0)
    l = lax.broadcasted_iota(jnp.int32, (_RB, 128), 1)
    xxi = (r & 3) * 128 + l
    yyi = hb * (_PB // _W) + (r >> 2)
    xx = xxi.astype(jnp.float32)
    yy = yyi.astype(jnp.float32)
    sm = jnp.where(jnp.abs(T) >= 1e-07, jnp.float32(0.0), jnp.float32(1e-06))
    Tt = T + sm
    v1 = X / Tt
    v2 = Y / Tt
    vgx = xx + (v1 - xx)
    vgy = yy + (v2 - yy)
    x0i = jnp.floor(vgx).astype(jnp.int32)
    y0i = jnp.floor(vgy).astype(jnp.int32)
    x0 = jnp.clip(x0i, 0, _W - 1)
    x1 = jnp.clip(x0i + 1, 0, _W - 1)
    y0 = jnp.clip(y0i, 0, _H - 1)
    y1 = jnp.clip(y0i + 1, 0, _H - 1)
    x0f = x0.astype(jnp.float32)
    x1f = x1.astype(jnp.float32)
    y0f = y0.astype(jnp.float32)
    y1f = y1.astype(jnp.float32)
    Xa = x1f - vgx
    Xc = vgx - x0f
    Ya = y1f - vgy
    Yb = vgy - y0f
    wa_ref[...] = Xa * Ya
    wb_ref[...] = Xa * Yb
    wc_ref[...] = Xc * Ya
    wd_ref[...] = Xc * Yb
    base = b * (_H * _W)
    i0_ref[...] = (base + y0 * _W) + x0
    i1_ref[...] = (base + y1 * _W) + x0
    sel_ref[...] = x1 - x0


def _stage_a(warped_r):
    i_sd = jax.ShapeDtypeStruct((_N // 128, 128), jnp.int32)
    f_sd = jax.ShapeDtypeStruct((_N // 128, 128), jnp.float32)
    out_spec = pl.BlockSpec((_RB, 128), lambda b, h: (b * _NB + h, 0))
    return pl.pallas_call(
        _stage_a_body,
        grid=(_B, _NB),
        in_specs=[pl.BlockSpec((1, 3, _RB, 128), lambda b, h: (b, 0, h, 0))],
        out_specs=[out_spec] * 7,
        out_shape=[i_sd, i_sd, i_sd, f_sd, f_sd, f_sd, f_sd],
    )(warped_r)


def _stage_b_body(table, i0, i1, g0, g1,
                  i0v0, i1v0, i0v1, i1v1, gv00, gv10, gv01, gv11,
                  isem0, isem1, gsem0, gsem1, osem0, osem1):
    wid = lax.axis_index("s") * 2 + lax.axis_index("c")
    idxbufs = ((i0v0, i1v0), (i0v1, i1v1))
    gvbufs = ((gv00, gv10), (gv01, gv11))
    isems = (isem0, isem1)
    gsems = (gsem0, gsem1)
    osems = (osem0, osem1)

    def in_handles(i, s):
        row0 = wid * (_PPW // 128) + i * _KG
        return [pltpu.make_async_copy(
                    src_ref.at[pl.ds(row0, _KG), :], dst, isems[s])
                for src_ref, dst in ((i0, idxbufs[s][0]), (i1, idxbufs[s][1]))]

    def gather_handles(s):
        hs = []
        for idx_v, dst in ((idxbufs[s][0], gvbufs[s][0]),
                           (idxbufs[s][1], gvbufs[s][1])):
            for k in range(_KG):
                hs.append(pltpu.make_async_copy(
                    table.at[idx_v.at[k]],
                    dst.at[pl.ds(k * 128, 128), :], gsems[s]))
        return hs

    def out_handles(i, s):
        p0 = wid * _PPW + i * _CH
        return [pltpu.make_async_copy(
                    gv, out_ref.at[pl.ds(p0, _CH), :], osems[s])
                for gv, out_ref in ((gvbufs[s][0], g0), (gvbufs[s][1], g1))]

    for h in in_handles(0, 0):
        h.start()

    def pair_body(j, _):
        for s in range(2):
            i = 2 * j + s
            other = 1 - s
            for h in in_handles(i, s):
                h.wait()

            @pl.when(j >= 1)
            def _():
                for h in out_handles(i - 2, s):
                    h.wait()

            for h in gather_handles(s):
                h.start()

            def drain_prev():
                for h in gather_handles(other):
                    h.wait()
                for h in out_handles(i - 1, other):
                    h.start()

            if s == 1:
                drain_prev()
            else:
                @pl.when(j >= 1)
                def _():
                    drain_prev()

            def start_next_in():
                for h in in_handles(i + 1, other):
                    h.start()

            if s == 0:
                start_next_in()
            else:
                @pl.when(j < _NCHUNK // 2 - 1)
                def _():
                    start_next_in()
        return 0

    lax.fori_loop(0, _NCHUNK // 2, pair_body, 0)

    last = _NCHUNK - 1
    s_last = last % 2
    for h in gather_handles(s_last):
        h.wait()
    for h in out_handles(last, s_last):
        h.start()
    for h in out_handles(last - 1, 1 - s_last):
        h.wait()
    for h in out_handles(last, s_last):
        h.wait()


def _stage_b(im8, i02, i12):
    mesh = plsc.VectorSubcoreMesh(core_axis_name="c", subcore_axis_name="s")
    g_sd = jax.ShapeDtypeStruct((_N, 8), jnp.float32)
    kern = pl.kernel(
        _stage_b_body,
        out_type=(g_sd, g_sd),
        mesh=mesh,
        scratch_types=[
            pltpu.VMEM((_KG, 128), jnp.int32),
            pltpu.VMEM((_KG, 128), jnp.int32),
            pltpu.VMEM((_KG, 128), jnp.int32),
            pltpu.VMEM((_KG, 128), jnp.int32),
            pltpu.VMEM((_CH, 8), jnp.float32),
            pltpu.VMEM((_CH, 8), jnp.float32),
            pltpu.VMEM((_CH, 8), jnp.float32),
            pltpu.VMEM((_CH, 8), jnp.float32),
            pltpu.SemaphoreType.DMA,
            pltpu.SemaphoreType.DMA,
            pltpu.SemaphoreType.DMA,
            pltpu.SemaphoreType.DMA,
            pltpu.SemaphoreType.DMA,
            pltpu.SemaphoreType.DMA,
        ],
        compiler_params=pltpu.CompilerParams(use_tc_tiling_on_sc=False),
    )
    return kern(im8, i02, i12)


def _stage_c_body(g0_ref, g1_ref, sel_ref,
                  wa_ref, wb_ref, wc_ref, wd_ref, out_ref):
    w_a = wa_ref[...]
    w_b = wb_ref[...]
    w_c = wc_ref[...]
    w_d = wd_ref[...]
    hi = sel_ref[...] > 0
    for ch in range(_C):
        i_a = g0_ref[ch]
        i_b = g1_ref[ch]
        i_c = jnp.where(hi, g0_ref[ch + 3], i_a)
        i_d = jnp.where(hi, g1_ref[ch + 3], i_b)
        out_ref[0, ch] = ((w_a * i_a + w_b * i_b) + w_c * i_c) + w_d * i_d


def _stage_c(g0p, g1p, sel, wa, wb, wc, wd):
    g_spec = pl.BlockSpec((6, _RB, 128), lambda b, h: (0, b * _NB + h, 0))
    w_spec = pl.BlockSpec((_RB, 128), lambda b, h: (b * _NB + h, 0))
    return pl.pallas_call(
        _stage_c_body,
        grid=(_B, _NB),
        in_specs=[g_spec, g_spec] + [w_spec] * 5,
        out_specs=pl.BlockSpec((1, _C, _RB, 128), lambda b, h: (b, 0, h, 0)),
        out_shape=jax.ShapeDtypeStruct((_B, _C, _NB * _RB, 128), jnp.float32),
    )(g0p, g1p, sel, wa, wb, wc, wd)


def kernel(src, H):
    b, c, h, w = src.shape
    xx = jnp.tile(jnp.arange(w)[None, :], (h, 1))
    yy = jnp.tile(jnp.arange(h)[:, None], (1, w))
    ones = jnp.ones((h, w), dtype=jnp.int32)
    g = jnp.stack([xx, yy, ones], axis=0).astype(jnp.float32)
    grid = jnp.broadcast_to(g[None], (b, 3, h, w))
    warped = jnp.einsum('bij,bjhw->bihw', H.reshape(b, 3, 3), grid)
    warped_r = warped.reshape(b, 3, _NB * _RB, 128)
    i0, i1, sel, wa, wb, wc, wd = _stage_a(warped_r)
    im_flat = src.transpose(0, 2, 3, 1).reshape(-1, c)
    shifted = jnp.concatenate([im_flat[1:], im_flat[:1]], axis=0)
    im8 = jnp.concatenate(
        [im_flat, shifted, jnp.zeros((_N, 2), jnp.float32)], axis=1)
    g0, g1 = _stage_b(im8, i0, i1)
    g0p = g0[:, :6].T.reshape(6, _N // 128, 128)
    g1p = g1[:, :6].T.reshape(6, _N // 128, 128)
    out = _stage_c(g0p, g1p, sel, wa, wb, wc, wd)
    return out.reshape(b, c, h, w)
